# Initial kernel scaffold; baseline (speedup 1.0000x reference)
import jax
import jax.numpy as jnp
from jax.experimental import pallas as pl


def kernel(edge_index, edge_label_index, emb):
    raise NotImplementedError("write your pallas kernel here")


# speedup vs baseline: 8.3107x; 8.3107x over previous
import functools

import jax
import jax.numpy as jnp
from jax import lax
from jax.experimental import pallas as pl
from jax.experimental.pallas import tpu as pltpu
from jax.experimental.pallas import tpu_sc as plsc

N = 100000
D = 32
E = 1600000
L = 65536
NLAYER = 3
ALPHA = 1.0 / (NLAYER + 1)

NC = 2
NS = 16
LANES = 16
K = 128

HALF = N // NC
DUMP = HALF
ACC_ROWS = 50176
ECHUNKS = E // K
NFULL = HALF // K
NREM = HALF - NFULL * K
ZCHUNKS = ACC_ROWS // K
PART_TILE = NFULL % NS

_F32 = jnp.float32
_I32 = jnp.int32


def _ntrips(limit, t):
    return (limit - t + NS - 1) // NS


def _fill_const(ref, value):
    v = jnp.full((LANES,), value, dtype=_F32)

    def body(r, _):
        ref[r, pl.ds(0, LANES)] = v
        ref[r, pl.ds(LANES, LANES)] = v
        return _

    lax.fori_loop(0, K, body, None)


def _zero_acc(t, acc_sh, zbuf):
    def body(i, _):
        zch = t + NS * i
        pltpu.sync_copy(zbuf, acc_sh.at[pl.ds(zch * K, K)])
        return _

    lax.fori_loop(0, _ntrips(ZCHUNKS, t), body, None)


def _edge_pass(c, t, col_hbm, acc_sh, cidx, lidx, row_hbm, y_hbm, ridx, rows,
               gsem, payload_is_ones=None):
    base_node = c * HALF

    def body(i, _):
        ch = t + NS * i
        eb = ch * K
        pltpu.sync_copy(col_hbm.at[pl.ds(eb, K)], cidx)
        for v in range(K // LANES):
            cv = cidx[pl.ds(v * LANES, LANES)]
            lc = cv - base_node
            ok = (lc >= 0) & (lc < HALF)
            lidx[pl.ds(v * LANES, LANES)] = jnp.where(
                ok, lc, jnp.full((LANES,), DUMP, dtype=_I32)
            )
        if payload_is_ones is None:
            pltpu.sync_copy(row_hbm.at[pl.ds(eb, K)], ridx)
            pltpu.async_copy(y_hbm.at[ridx], rows, gsem).wait()
            pltpu.sync_copy(rows, acc_sh.at[lidx], add=True)
        else:
            pltpu.sync_copy(payload_is_ones, acc_sh.at[lidx], add=True)
        return _

    lax.fori_loop(0, _ntrips(ECHUNKS, t), body, None)


def _deg_body(col_hbm, deg_hbm, cidx, lidx, ones_b, acc_sh):
    c = lax.axis_index("c")
    t = lax.axis_index("s")
    _fill_const(ones_b, 0.0)
    _zero_acc(t, acc_sh, ones_b)
    _fill_const(ones_b, 1.0)
    plsc.subcore_barrier()

    _edge_pass(c, t, col_hbm, acc_sh, cidx, lidx,
               None, None, None, None, None, payload_is_ones=ones_b)
    plsc.subcore_barrier()

    base_node = c * HALF

    def node_chunk(nch, nrows):
        lbase = nch * K
        pltpu.sync_copy(acc_sh.at[pl.ds(lbase, nrows)],
                        deg_hbm.at[pl.ds(base_node + lbase, nrows)])

    def body(i, _):
        node_chunk(t + NS * i, K)
        return _

    lax.fori_loop(0, _ntrips(NFULL, t), body, None)

    @pl.when(t == PART_TILE)
    def _():
        node_chunk(NFULL, NREM)


def _scale_body(deg_ref, x_ref, s_ref, y_ref, o_ref):
    d = deg_ref[...]
    s = jnp.where(d > 0.5, lax.rsqrt(jnp.maximum(d, 1e-12)), 0.0)
    x = x_ref[...]
    s_ref[...] = s
    y_ref[...] = s * x
    o_ref[...] = ALPHA * x


def _layer_body(row_hbm, col_hbm, y_hbm, s_hbm, out_hbm,
                y_out, out_out,
                cidx, lidx, ridx, rows, zbuf, acc_t, s_t, o_t, y_t,
                gsem, acc_sh):
    c = lax.axis_index("c")
    t = lax.axis_index("s")
    _fill_const(zbuf, 0.0)
    _zero_acc(t, acc_sh, zbuf)
    plsc.subcore_barrier()

    _edge_pass(c, t, col_hbm, acc_sh, cidx, lidx, row_hbm, y_hbm, ridx, rows,
               gsem)
    plsc.subcore_barrier()

    base_node = c * HALF

    def node_chunk(nch, nrows):
        lbase = nch * K
        gbase = base_node + lbase
        pltpu.sync_copy(acc_sh.at[pl.ds(lbase, nrows)],
                        acc_t.at[pl.ds(0, nrows)])
        pltpu.sync_copy(s_hbm.at[pl.ds(gbase, nrows)], s_t.at[pl.ds(0, nrows)])
        pltpu.sync_copy(out_hbm.at[pl.ds(gbase, nrows)],
                        o_t.at[pl.ds(0, nrows)])

        def row_body(r, _):
            for h in range(2):
                sl = pl.ds(h * LANES, LANES)
                sv = s_t[r, sl]
                sa = sv * acc_t[r, sl]
                y_t[r, sl] = sv * sa
                o_t[r, sl] = o_t[r, sl] + ALPHA * sa
            return _

        lax.fori_loop(0, nrows, row_body, None)
        pltpu.sync_copy(y_t.at[pl.ds(0, nrows)], y_out.at[pl.ds(gbase, nrows)])
        pltpu.sync_copy(o_t.at[pl.ds(0, nrows)],
                        out_out.at[pl.ds(gbase, nrows)])

    def body(i, _):
        node_chunk(t + NS * i, K)
        return _

    lax.fori_loop(0, _ntrips(NFULL, t), body, None)

    @pl.when(t == PART_TILE)
    def _():
        node_chunk(NFULL, NREM)


def _rank_body(a_hbm, b_hbm, out_hbm, rank_hbm,
               aidx, bidx, ra, rb, rk, gsem):
    c = lax.axis_index("c")
    t = lax.axis_index("s")
    w = t * NC + c
    chunks_per_w = L // K // (NC * NS)

    for i in range(chunks_per_w):
        base = (w * chunks_per_w + i) * K
        pltpu.sync_copy(a_hbm.at[pl.ds(base, K)], aidx)
        pltpu.sync_copy(b_hbm.at[pl.ds(base, K)], bidx)
        pltpu.async_copy(out_hbm.at[aidx], ra, gsem).wait()
        pltpu.async_copy(out_hbm.at[bidx], rb, gsem).wait()

        lane_id = lax.iota(_I32, LANES)

        def group_body(g, _):
            res = jnp.zeros((LANES,), dtype=_F32)
            for j in range(LANES):
                p = g * LANES + j
                pr = (ra[p, pl.ds(0, LANES)] * rb[p, pl.ds(0, LANES)]
                      + ra[p, pl.ds(LANES, LANES)]
                      * rb[p, pl.ds(LANES, LANES)])
                res = jnp.where(lane_id == j, jnp.sum(pr), res)
            rk[pl.ds(g * LANES, LANES)] = res
            return _

        lax.fori_loop(0, K // LANES, group_body, None)
        pltpu.sync_copy(rk, rank_hbm.at[pl.ds(base, K)])


_MESH = plsc.VectorSubcoreMesh(core_axis_name="c", subcore_axis_name="s")

_deg = pl.kernel(
    _deg_body,
    out_type=jax.ShapeDtypeStruct((N, D), _F32),
    mesh=_MESH,
    compiler_params=pltpu.CompilerParams(use_tc_tiling_on_sc=False, needs_layout_passes=False),
    scratch_types=(
        pltpu.VMEM((K,), _I32),
        pltpu.VMEM((K,), _I32),
        pltpu.VMEM((K, D), _F32),
        pltpu.VMEM_SHARED((ACC_ROWS, D), _F32),
    ),
)

_SCALE_BLK = 1000

_scale = pl.pallas_call(
    _scale_body,
    grid=(N // _SCALE_BLK,),
    in_specs=[
        pl.BlockSpec((_SCALE_BLK, D), lambda i: (i, 0)),
        pl.BlockSpec((_SCALE_BLK, D), lambda i: (i, 0)),
    ],
    out_specs=[
        pl.BlockSpec((_SCALE_BLK, D), lambda i: (i, 0)),
        pl.BlockSpec((_SCALE_BLK, D), lambda i: (i, 0)),
        pl.BlockSpec((_SCALE_BLK, D), lambda i: (i, 0)),
    ],
    out_shape=(
        jax.ShapeDtypeStruct((N, D), _F32),
        jax.ShapeDtypeStruct((N, D), _F32),
        jax.ShapeDtypeStruct((N, D), _F32),
    ),
)

_layer = pl.kernel(
    _layer_body,
    out_type=(
        jax.ShapeDtypeStruct((N, D), _F32),
        jax.ShapeDtypeStruct((N, D), _F32),
    ),
    mesh=_MESH,
    compiler_params=pltpu.CompilerParams(use_tc_tiling_on_sc=False, needs_layout_passes=False),
    scratch_types=(
        pltpu.VMEM((K,), _I32),
        pltpu.VMEM((K,), _I32),
        pltpu.VMEM((K,), _I32),
        pltpu.VMEM((K, D), _F32),
        pltpu.VMEM((K, D), _F32),
        pltpu.VMEM((K, D), _F32),
        pltpu.VMEM((K, D), _F32),
        pltpu.VMEM((K, D), _F32),
        pltpu.VMEM((K, D), _F32),
        pltpu.SemaphoreType.DMA,
        pltpu.VMEM_SHARED((ACC_ROWS, D), _F32),
    ),
)

_rank = pl.kernel(
    _rank_body,
    out_type=jax.ShapeDtypeStruct((L,), _F32),
    mesh=_MESH,
    compiler_params=pltpu.CompilerParams(use_tc_tiling_on_sc=False, needs_layout_passes=False),
    scratch_types=(
        pltpu.VMEM((K,), _I32),
        pltpu.VMEM((K,), _I32),
        pltpu.VMEM((K, D), _F32),
        pltpu.VMEM((K, D), _F32),
        pltpu.VMEM((K,), _F32),
        pltpu.SemaphoreType.DMA,
    ),
)


def kernel(edge_index, edge_label_index, emb):
    row = edge_index[0]
    col = edge_index[1]
    deg = _deg(col)
    s, y, out = _scale(deg, emb)
    for _ in range(NLAYER):
        y, out = _layer(row, col, y, s, out)
    return _rank(edge_label_index[0], edge_label_index[1], out)


# speedup vs baseline: 10.0841x; 1.2134x over previous
import functools

import jax
import jax.numpy as jnp
from jax import lax
from jax.experimental import pallas as pl
from jax.experimental.pallas import tpu as pltpu
from jax.experimental.pallas import tpu_sc as plsc

N = 100000
D = 32
E = 1600000
L = 65536
NLAYER = 3
ALPHA = 1.0 / (NLAYER + 1)

NC = 2
NS = 16
LANES = 16
K = 128

HALF = N // NC
DUMP = HALF
ACC_ROWS = 50176
ECHUNKS = E // K
NFULL = HALF // K
NREM = HALF - NFULL * K
ZCHUNKS = ACC_ROWS // K
DEGW = 8
PART_TILE = NFULL % NS

_F32 = jnp.float32
_I32 = jnp.int32


def _ntrips(limit, t):
    return (limit - t + NS - 1) // NS


def _fill_const(ref, value):
    v = jnp.full((LANES,), value, dtype=_F32)

    def body(r, _):
        ref[r, pl.ds(0, LANES)] = v
        ref[r, pl.ds(LANES, LANES)] = v
        return _

    lax.fori_loop(0, K, body, None)


def _zero_acc(t, acc_sh, zbuf):
    def body(i, _):
        zch = t + NS * i
        pltpu.sync_copy(zbuf, acc_sh.at[pl.ds(zch * K, K)])
        return _

    lax.fori_loop(0, _ntrips(ZCHUNKS, t), body, None)


def _edge_pass(c, t, col_hbm, acc_sh, cidx, lidx, row_hbm, y_hbm, ridx, rows,
               gsem, payload_is_ones=None):
    base_node = c * HALF

    def body(i, _):
        ch = t + NS * i
        eb = ch * K
        pltpu.sync_copy(col_hbm.at[pl.ds(eb, K)], cidx)
        for v in range(K // LANES):
            cv = cidx[pl.ds(v * LANES, LANES)]
            lc = cv - base_node
            ok = (lc >= 0) & (lc < HALF)
            lidx[pl.ds(v * LANES, LANES)] = jnp.where(
                ok, lc, jnp.full((LANES,), DUMP, dtype=_I32)
            )
        if payload_is_ones is None:
            pltpu.sync_copy(row_hbm.at[pl.ds(eb, K)], ridx)
            pltpu.async_copy(y_hbm.at[ridx], rows, gsem).wait()
            pltpu.sync_copy(rows, acc_sh.at[lidx], add=True)
        else:
            pltpu.sync_copy(payload_is_ones, acc_sh.at[lidx], add=True)
        return _

    lax.fori_loop(0, _ntrips(ECHUNKS, t), body, None)


def _deg_body(col_hbm, ones_hbm, zeros_hbm, deg_hbm, cidx, lidx, ones_b,
              zeros_b, acc_sh):
    c = lax.axis_index("c")
    t = lax.axis_index("s")
    pltpu.sync_copy(ones_hbm, ones_b)
    pltpu.sync_copy(zeros_hbm, zeros_b)
    _zero_acc(t, acc_sh, zeros_b)
    plsc.subcore_barrier()

    _edge_pass(c, t, col_hbm, acc_sh, cidx, lidx,
               None, None, None, None, None, payload_is_ones=ones_b)
    plsc.subcore_barrier()

    base_node = c * HALF

    def node_chunk(nch, nrows):
        lbase = nch * K
        pltpu.sync_copy(acc_sh.at[pl.ds(lbase, nrows)],
                        deg_hbm.at[pl.ds(base_node + lbase, nrows)])

    def body(i, _):
        node_chunk(t + NS * i, K)
        return _

    lax.fori_loop(0, _ntrips(NFULL, t), body, None)

    @pl.when(t == PART_TILE)
    def _():
        node_chunk(NFULL, NREM)


def _scale_body(deg_ref, x_ref, s_ref, y_ref, o_ref):
    d = jnp.broadcast_to(deg_ref[...][:, :1], (deg_ref.shape[0], D))
    s = jnp.where(d > 0.5, lax.rsqrt(jnp.maximum(d, 1e-12)), 0.0)
    x = x_ref[...]
    s_ref[...] = s
    y_ref[...] = s * x
    o_ref[...] = ALPHA * x


def _layer_body(row_hbm, col_hbm, y_hbm, s_hbm, out_hbm,
                y_out, out_out,
                cidx0, cidx1, lidx0, lidx1, ridx0, ridx1, rows0, rows1,
                acc_t, s_t, o_t, y_t,
                gsem0, gsem1, ssem0, ssem1, acc_sh):
    c = lax.axis_index("c")
    t = lax.axis_index("s")
    _fill_const(y_t, 0.0)
    _zero_acc(t, acc_sh, y_t)
    plsc.subcore_barrier()

    base_node = c * HALF
    cidx = (cidx0, cidx1)
    lidx = (lidx0, lidx1)
    ridx = (ridx0, ridx1)
    rows = (rows0, rows1)
    gsem = (gsem0, gsem1)
    ssem = (ssem0, ssem1)

    def stage(trip, sl):
        ch = t + NS * trip
        valid = ch < ECHUNKS
        eb = jnp.minimum(ch, ECHUNKS - 1) * K
        pltpu.sync_copy(row_hbm.at[pl.ds(eb, K)], ridx[sl])
        pltpu.sync_copy(col_hbm.at[pl.ds(eb, K)], cidx[sl])
        off = jnp.where(valid, 0, 2 * HALF)
        for v in range(K // LANES):
            cv = cidx[sl][pl.ds(v * LANES, LANES)]
            lc = cv - base_node + off
            ok = (lc >= 0) & (lc < HALF)
            lidx[sl][pl.ds(v * LANES, LANES)] = jnp.where(
                ok, lc, jnp.full((LANES,), DUMP, dtype=_I32)
            )

    def swait(sl):
        pltpu.make_async_copy(rows[sl], acc_sh.at[lidx[sl]], ssem[sl]).wait()

    def body(g, _):
        @pl.when(g > 0)
        def _():
            swait(0)
            swait(1)

        stage(2 * g, 0)
        d0 = pltpu.async_copy(y_hbm.at[ridx[0]], rows[0], gsem[0])
        stage(2 * g + 1, 1)
        d1 = pltpu.async_copy(y_hbm.at[ridx[1]], rows[1], gsem[1])
        d0.wait()
        pltpu.async_copy(rows[0], acc_sh.at[lidx[0]], ssem[0], add=True)
        d1.wait()
        pltpu.async_copy(rows[1], acc_sh.at[lidx[1]], ssem[1], add=True)
        return _

    lax.fori_loop(0, (ECHUNKS + 2 * NS - 1) // (2 * NS), body, None)
    swait(0)
    swait(1)
    plsc.subcore_barrier()

    base_node = c * HALF

    def node_chunk(nch, nrows):
        lbase = nch * K
        gbase = base_node + lbase
        pltpu.sync_copy(acc_sh.at[pl.ds(lbase, nrows)],
                        acc_t.at[pl.ds(0, nrows)])
        pltpu.sync_copy(s_hbm.at[pl.ds(gbase, nrows)], s_t.at[pl.ds(0, nrows)])
        pltpu.sync_copy(out_hbm.at[pl.ds(gbase, nrows)],
                        o_t.at[pl.ds(0, nrows)])

        def row_body(r, _):
            for h in range(2):
                sl = pl.ds(h * LANES, LANES)
                sv = s_t[r, sl]
                sa = sv * acc_t[r, sl]
                y_t[r, sl] = sv * sa
                o_t[r, sl] = o_t[r, sl] + ALPHA * sa
            return _

        lax.fori_loop(0, nrows, row_body, None)
        pltpu.sync_copy(y_t.at[pl.ds(0, nrows)], y_out.at[pl.ds(gbase, nrows)])
        pltpu.sync_copy(o_t.at[pl.ds(0, nrows)],
                        out_out.at[pl.ds(gbase, nrows)])

    def body(i, _):
        node_chunk(t + NS * i, K)
        return _

    lax.fori_loop(0, _ntrips(NFULL, t), body, None)

    @pl.when(t == PART_TILE)
    def _():
        node_chunk(NFULL, NREM)


def _rank_body(a_hbm, b_hbm, out_hbm, rank_hbm,
               aidx, bidx, ra, rb, rk, gsem):
    c = lax.axis_index("c")
    t = lax.axis_index("s")
    w = t * NC + c
    chunks_per_w = L // K // (NC * NS)

    for i in range(chunks_per_w):
        base = (w * chunks_per_w + i) * K
        pltpu.sync_copy(a_hbm.at[pl.ds(base, K)], aidx)
        pltpu.sync_copy(b_hbm.at[pl.ds(base, K)], bidx)
        pltpu.async_copy(out_hbm.at[aidx], ra, gsem).wait()
        pltpu.async_copy(out_hbm.at[bidx], rb, gsem).wait()

        lane_id = lax.iota(_I32, LANES)

        def group_body(g, _):
            res = jnp.zeros((LANES,), dtype=_F32)
            for j in range(LANES):
                p = g * LANES + j
                pr = (ra[p, pl.ds(0, LANES)] * rb[p, pl.ds(0, LANES)]
                      + ra[p, pl.ds(LANES, LANES)]
                      * rb[p, pl.ds(LANES, LANES)])
                res = jnp.where(lane_id == j, jnp.sum(pr), res)
            rk[pl.ds(g * LANES, LANES)] = res
            return _

        lax.fori_loop(0, K // LANES, group_body, None)
        pltpu.sync_copy(rk, rank_hbm.at[pl.ds(base, K)])


_MESH = plsc.VectorSubcoreMesh(core_axis_name="c", subcore_axis_name="s")

_deg = pl.kernel(
    _deg_body,
    out_type=jax.ShapeDtypeStruct((N, DEGW), _F32),
    mesh=_MESH,
    compiler_params=pltpu.CompilerParams(use_tc_tiling_on_sc=False, needs_layout_passes=False),
    scratch_types=(
        pltpu.VMEM((K,), _I32),
        pltpu.VMEM((K,), _I32),
        pltpu.VMEM((K, DEGW), _F32),
        pltpu.VMEM((K, DEGW), _F32),
        pltpu.VMEM_SHARED((ACC_ROWS, DEGW), _F32),
    ),
)

_SCALE_BLK = 1000

_scale = pl.pallas_call(
    _scale_body,
    grid=(N // _SCALE_BLK,),
    in_specs=[
        pl.BlockSpec((_SCALE_BLK, DEGW), lambda i: (i, 0)),
        pl.BlockSpec((_SCALE_BLK, D), lambda i: (i, 0)),
    ],
    out_specs=[
        pl.BlockSpec((_SCALE_BLK, D), lambda i: (i, 0)),
        pl.BlockSpec((_SCALE_BLK, D), lambda i: (i, 0)),
        pl.BlockSpec((_SCALE_BLK, D), lambda i: (i, 0)),
    ],
    out_shape=(
        jax.ShapeDtypeStruct((N, D), _F32),
        jax.ShapeDtypeStruct((N, D), _F32),
        jax.ShapeDtypeStruct((N, D), _F32),
    ),
)

_layer = pl.kernel(
    _layer_body,
    out_type=(
        jax.ShapeDtypeStruct((N, D), _F32),
        jax.ShapeDtypeStruct((N, D), _F32),
    ),
    mesh=_MESH,
    compiler_params=pltpu.CompilerParams(use_tc_tiling_on_sc=False, needs_layout_passes=False),
    scratch_types=(
        pltpu.VMEM((K,), _I32),
        pltpu.VMEM((K,), _I32),
        pltpu.VMEM((K,), _I32),
        pltpu.VMEM((K,), _I32),
        pltpu.VMEM((K,), _I32),
        pltpu.VMEM((K,), _I32),
        pltpu.VMEM((K, D), _F32),
        pltpu.VMEM((K, D), _F32),
        pltpu.VMEM((K, D), _F32),
        pltpu.VMEM((K, D), _F32),
        pltpu.VMEM((K, D), _F32),
        pltpu.VMEM((K, D), _F32),
        pltpu.SemaphoreType.DMA,
        pltpu.SemaphoreType.DMA,
        pltpu.SemaphoreType.DMA,
        pltpu.SemaphoreType.DMA,
        pltpu.VMEM_SHARED((ACC_ROWS, D), _F32),
    ),
)

_rank = pl.kernel(
    _rank_body,
    out_type=jax.ShapeDtypeStruct((L,), _F32),
    mesh=_MESH,
    compiler_params=pltpu.CompilerParams(use_tc_tiling_on_sc=False, needs_layout_passes=False),
    scratch_types=(
        pltpu.VMEM((K,), _I32),
        pltpu.VMEM((K,), _I32),
        pltpu.VMEM((K, D), _F32),
        pltpu.VMEM((K, D), _F32),
        pltpu.VMEM((K,), _F32),
        pltpu.SemaphoreType.DMA,
    ),
)


def kernel(edge_index, edge_label_index, emb):
    row = edge_index[0]
    col = edge_index[1]
    deg = _deg(col, jnp.ones((K, DEGW), _F32), jnp.zeros((K, DEGW), _F32))
    s, y, out = _scale(deg, emb)
    for _ in range(NLAYER):
        y, out = _layer(row, col, y, s, out)
    return _rank(edge_label_index[0], edge_label_index[1], out)
